# Initial kernel scaffold; baseline (speedup 1.0000x reference)
import jax
import jax.numpy as jnp
from jax.experimental import pallas as pl


def kernel(x, edge_index, W1, b1, W2, b2):
    raise NotImplementedError("write your pallas kernel here")


# speedup vs baseline: 5.2420x; 5.2420x over previous
import functools

import jax
import jax.numpy as jnp
from jax import lax
from jax.experimental import pallas as pl
from jax.experimental.pallas import tpu as pltpu
from jax.experimental.pallas import tpu_sc as plsc

N = 10000
E = 320000
F_IN = 128
H = 256
C = 64

NC = 2
NS = 16
K = 128
CHUNKS = 80
E_PAD = NC * NS * CHUNKS * K
RPT = 632
NROWS = NS * RPT
DUMMY = 10008


def _make_seg_sum(F):
    @functools.partial(
        pl.kernel,
        out_type=jax.ShapeDtypeStruct((NC, NROWS, F), jnp.float32),
        mesh=plsc.VectorSubcoreMesh(core_axis_name="c", subcore_axis_name="s"),
        scratch_types=[
            pltpu.VMEM((CHUNKS, K), jnp.int32),
            pltpu.VMEM((2, K), jnp.int32),
            pltpu.VMEM((2, K), jnp.int32),
            pltpu.VMEM((K, F), jnp.float32),
            pltpu.VMEM((K, F), jnp.float32),
            pltpu.VMEM_SHARED((NROWS, F), jnp.float32),
            pltpu.SemaphoreType.DMA,
            pltpu.SemaphoreType.DMA,
        ],
        compiler_params=pltpu.CompilerParams(use_tc_tiling_on_sc=False),
    )
    def seg_sum(table, edges, zeros, out, edge_v, sidx, didx, rows0, rows1,
                acc, sem0, sem1):
        cid = lax.axis_index("c")
        sid = lax.axis_index("s")
        r0 = sid * RPT

        pltpu.sync_copy(edges.at[cid].at[sid], edge_v)
        pltpu.sync_copy(zeros, acc.at[pl.ds(r0, RPT)])

        def unpack(c, b):
            for j in range(K // 16):
                p = edge_v.at[c][pl.ds(16 * j, 16)]
                sidx.at[b][pl.ds(16 * j, 16)] = lax.bitwise_and(p, 0xFFFF)
                didx.at[b][pl.ds(16 * j, 16)] = lax.shift_right_logical(p, 16)

        plsc.subcore_barrier()

        unpack(0, 0)
        pltpu.async_copy(table.at[sidx.at[0]], rows0, sem0)

        def body(i, carry):
            a = 2 * i
            unpack(a + 1, 1)
            pltpu.async_copy(table.at[sidx.at[1]], rows1, sem1)
            pltpu.make_async_copy(table.at[sidx.at[0]], rows0, sem0).wait()
            pltpu.sync_copy(rows0, acc.at[didx.at[0]], add=True)

            @pl.when(i < CHUNKS // 2 - 1)
            def _():
                unpack(a + 2, 0)
                pltpu.async_copy(table.at[sidx.at[0]], rows0, sem0)

            pltpu.make_async_copy(table.at[sidx.at[1]], rows1, sem1).wait()
            pltpu.sync_copy(rows1, acc.at[didx.at[1]], add=True)
            return carry

        lax.fori_loop(0, CHUNKS // 2, body, 0)
        plsc.subcore_barrier()

        pltpu.sync_copy(acc.at[pl.ds(r0, RPT)],
                        out.at[cid].at[pl.ds(r0, RPT)])

    return seg_sum


_seg_sum_128 = _make_seg_sum(F_IN)
_seg_sum_64 = _make_seg_sum(C)


def _mlp_body(x_ref, a_ref, w1_ref, b1_ref, w2_ref, o_ref):
    h = x_ref[...] + a_ref[0] + a_ref[1]
    h1 = jnp.dot(h, w1_ref[...], preferred_element_type=jnp.float32)
    h1 = jnp.maximum(h1 + b1_ref[...], 0.0)
    o_ref[...] = jnp.dot(h1, w2_ref[...], preferred_element_type=jnp.float32)


def _log_softmax_body(v_ref, a_ref, b2_ref, o_ref):
    h2 = v_ref[...] + a_ref[0] + a_ref[1] + b2_ref[...]
    m = jnp.max(h2, axis=1, keepdims=True)
    s = jnp.sum(jnp.exp(h2 - m), axis=1, keepdims=True)
    o_ref[...] = h2 - m - jnp.log(s)


_BR = 1000


def _mlp(x, aggr, W1, b1, W2):
    return pl.pallas_call(
        _mlp_body,
        grid=(N // _BR,),
        in_specs=[
            pl.BlockSpec((_BR, F_IN), lambda i: (i, 0)),
            pl.BlockSpec((NC, _BR, F_IN), lambda i: (0, i, 0)),
            pl.BlockSpec((F_IN, H), lambda i: (0, 0)),
            pl.BlockSpec((1, H), lambda i: (0, 0)),
            pl.BlockSpec((H, C), lambda i: (0, 0)),
        ],
        out_specs=pl.BlockSpec((_BR, C), lambda i: (i, 0)),
        out_shape=jax.ShapeDtypeStruct((N, C), jnp.float32),
    )(x, aggr, W1, b1.reshape(1, H), W2)


def _log_softmax(v, aggr, b2):
    return pl.pallas_call(
        _log_softmax_body,
        grid=(N // _BR,),
        in_specs=[
            pl.BlockSpec((_BR, C), lambda i: (i, 0)),
            pl.BlockSpec((NC, _BR, C), lambda i: (0, i, 0)),
            pl.BlockSpec((1, C), lambda i: (0, 0)),
        ],
        out_specs=pl.BlockSpec((_BR, C), lambda i: (i, 0)),
        out_shape=jax.ShapeDtypeStruct((N, C), jnp.float32),
    )(v, aggr, b2.reshape(1, C))


def kernel(x, edge_index, W1, b1, W2, b2):
    src = edge_index[0].astype(jnp.int32)
    dst = edge_index[1].astype(jnp.int32)
    pad = E_PAD - E
    packed = jnp.left_shift(dst, 16) | src
    packed = jnp.concatenate([packed, jnp.full((pad,), DUMMY << 16, jnp.int32)])
    edges = packed.reshape(NC, NS, CHUNKS, K)

    z128 = jnp.zeros((RPT, F_IN), jnp.float32)
    z64 = jnp.zeros((RPT, C), jnp.float32)

    aggr1 = _seg_sum_128(x, edges, z128)
    v = _mlp(x, aggr1, W1, b1, W2)
    aggr2 = _seg_sum_64(v, edges, z64)
    return _log_softmax(v, aggr2, b2)


# speedup vs baseline: 5.7006x; 1.0875x over previous
import functools

import jax
import jax.numpy as jnp
from jax import lax
from jax.experimental import pallas as pl
from jax.experimental.pallas import tpu as pltpu
from jax.experimental.pallas import tpu_sc as plsc

N = 10000
E = 320000
F_IN = 128
H = 256
C = 64

NC = 2
NS = 16
K = 128
CH0 = 128
CH1 = 32
CHMAX = max(CH0, CH1)
TOTC = CH0 + CH1
E_PAD = NS * (CH0 + CH1) * K
RPT = 632
NROWS = NS * RPT
DUMMY = 10008


def _make_seg_sum(F):
    @functools.partial(
        pl.kernel,
        out_type=jax.ShapeDtypeStruct((NC, NROWS, F), jnp.float32),
        mesh=plsc.VectorSubcoreMesh(core_axis_name="c", subcore_axis_name="s"),
        scratch_types=[
            pltpu.VMEM((CHMAX, K), jnp.int32),
            pltpu.VMEM((2, K), jnp.int32),
            pltpu.VMEM((2, K), jnp.int32),
            pltpu.VMEM((K, F), jnp.float32),
            pltpu.VMEM((K, F), jnp.float32),
            pltpu.VMEM_SHARED((NROWS, F), jnp.float32),
            pltpu.SemaphoreType.DMA,
            pltpu.SemaphoreType.DMA,
        ],
        compiler_params=pltpu.CompilerParams(use_tc_tiling_on_sc=False),
    )
    def seg_sum(table, edges, zeros, out, edge_v, sidx, didx, rows0, rows1,
                acc, sem0, sem1):
        cid = lax.axis_index("c")
        sid = lax.axis_index("s")
        r0 = sid * RPT
        start = jnp.minimum(cid * CH0, TOTC - CHMAX)
        coff = cid * CH0 - start
        nch = jnp.where(cid == 0, CH0, CH1)

        pltpu.sync_copy(edges.at[sid].at[pl.ds(start, CHMAX)], edge_v)
        for t in range(7):
            pltpu.sync_copy(zeros, acc.at[pl.ds(r0 + 80 * t, 80)])
        pltpu.sync_copy(zeros.at[pl.ds(0, 72)], acc.at[pl.ds(r0 + 560, 72)])

        def unpack(c, b):
            for j in range(K // 16):
                p = edge_v.at[c + coff][pl.ds(16 * j, 16)]
                sidx.at[b][pl.ds(16 * j, 16)] = lax.bitwise_and(p, 0xFFFF)
                didx.at[b][pl.ds(16 * j, 16)] = lax.shift_right_logical(p, 16)

        plsc.subcore_barrier()

        @pl.when(nch > 0)
        def _():
            unpack(0, 0)
            pltpu.async_copy(table.at[sidx.at[0]], rows0, sem0)

        def body(i, carry):
            a = 2 * i
            unpack(a + 1, 1)
            pltpu.async_copy(table.at[sidx.at[1]], rows1, sem1)
            pltpu.make_async_copy(table.at[sidx.at[0]], rows0, sem0).wait()
            pltpu.sync_copy(rows0, acc.at[didx.at[0]], add=True)

            @pl.when(i < nch // 2 - 1)
            def _():
                unpack(a + 2, 0)
                pltpu.async_copy(table.at[sidx.at[0]], rows0, sem0)

            pltpu.make_async_copy(table.at[sidx.at[1]], rows1, sem1).wait()
            pltpu.sync_copy(rows1, acc.at[didx.at[1]], add=True)
            return carry

        lax.fori_loop(0, nch // 2, body, 0)
        plsc.subcore_barrier()

        pltpu.sync_copy(acc.at[pl.ds(r0, RPT)],
                        out.at[cid].at[pl.ds(r0, RPT)])

    return seg_sum


_seg_sum_128 = _make_seg_sum(F_IN)
_seg_sum_64 = _make_seg_sum(C)


def _mlp_body(x_ref, a_ref, w1_ref, b1_ref, w2_ref, o_ref):
    h = x_ref[...] + a_ref[0] + a_ref[1]
    h1 = jnp.dot(h, w1_ref[...], preferred_element_type=jnp.float32)
    h1 = jnp.maximum(h1 + b1_ref[...], 0.0)
    o_ref[...] = jnp.dot(h1, w2_ref[...], preferred_element_type=jnp.float32)


def _log_softmax_body(v_ref, a_ref, b2_ref, o_ref):
    h2 = v_ref[...] + a_ref[0] + a_ref[1] + b2_ref[...]
    m = jnp.max(h2, axis=1, keepdims=True)
    s = jnp.sum(jnp.exp(h2 - m), axis=1, keepdims=True)
    o_ref[...] = h2 - m - jnp.log(s)


_BR = 1000


def _mlp(x, aggr, W1, b1, W2):
    return pl.pallas_call(
        _mlp_body,
        grid=(N // _BR,),
        in_specs=[
            pl.BlockSpec((_BR, F_IN), lambda i: (i, 0)),
            pl.BlockSpec((NC, _BR, F_IN), lambda i: (0, i, 0)),
            pl.BlockSpec((F_IN, H), lambda i: (0, 0)),
            pl.BlockSpec((1, H), lambda i: (0, 0)),
            pl.BlockSpec((H, C), lambda i: (0, 0)),
        ],
        out_specs=pl.BlockSpec((_BR, C), lambda i: (i, 0)),
        out_shape=jax.ShapeDtypeStruct((N, C), jnp.float32),
    )(x, aggr, W1, b1.reshape(1, H), W2)


def _log_softmax(v, aggr, b2):
    return pl.pallas_call(
        _log_softmax_body,
        grid=(N // _BR,),
        in_specs=[
            pl.BlockSpec((_BR, C), lambda i: (i, 0)),
            pl.BlockSpec((NC, _BR, C), lambda i: (0, i, 0)),
            pl.BlockSpec((1, C), lambda i: (0, 0)),
        ],
        out_specs=pl.BlockSpec((_BR, C), lambda i: (i, 0)),
        out_shape=jax.ShapeDtypeStruct((N, C), jnp.float32),
    )(v, aggr, b2.reshape(1, C))


def kernel(x, edge_index, W1, b1, W2, b2):
    src = edge_index[0].astype(jnp.int32)
    dst = edge_index[1].astype(jnp.int32)
    pad = E_PAD - E
    packed = jnp.left_shift(dst, 16) | src
    packed = jnp.concatenate([packed, jnp.full((pad,), DUMMY << 16, jnp.int32)])
    edges = packed.reshape(NS, TOTC, K)

    z128 = jnp.zeros((80, F_IN), jnp.float32)
    z64 = jnp.zeros((80, C), jnp.float32)

    aggr1 = _seg_sum_128(x, edges, z128)
    v = _mlp(x, aggr1, W1, b1, W2)
    aggr2 = _seg_sum_64(v, edges, z64)
    return _log_softmax(v, aggr2, b2)


# speedup vs baseline: 6.4967x; 1.1397x over previous
import functools

import jax
import jax.numpy as jnp
from jax import lax
from jax.experimental import pallas as pl
from jax.experimental.pallas import tpu as pltpu
from jax.experimental.pallas import tpu_sc as plsc

N = 10000
E = 320000
F_IN = 128
H = 256
C = 64

NC = 2
NS = 16
K = 128
CH0 = 128
CH1 = 32
CHMAX = max(CH0, CH1)
TOTC = CH0 + CH1
E_PAD = NS * (CH0 + CH1) * K
RPT = 632
NROWS = NS * RPT
DUMMY = 10008


def _make_seg_sum(F, spmem_table=False):
    scratch = [
        pltpu.VMEM((CHMAX, K), jnp.int32),
        pltpu.VMEM((2, K), jnp.int32),
        pltpu.VMEM((2, K), jnp.int32),
        pltpu.VMEM((K, F), jnp.float32),
        pltpu.VMEM((K, F), jnp.float32),
        pltpu.VMEM_SHARED((NROWS, F), jnp.float32),
        pltpu.SemaphoreType.DMA,
        pltpu.SemaphoreType.DMA,
    ]
    if spmem_table:
        scratch.append(pltpu.VMEM_SHARED((NROWS, F), jnp.float32))

    @functools.partial(
        pl.kernel,
        out_type=jax.ShapeDtypeStruct((NC, NROWS, F), jnp.float32),
        mesh=plsc.VectorSubcoreMesh(core_axis_name="c", subcore_axis_name="s"),
        scratch_types=scratch,
        compiler_params=pltpu.CompilerParams(use_tc_tiling_on_sc=False),
    )
    def seg_sum(table, edges, zeros, out, edge_v, sidx, didx, rows0, rows1,
                acc, sem0, sem1, *maybe_tbl):
        cid = lax.axis_index("c")
        sid = lax.axis_index("s")
        r0 = sid * RPT
        start = jnp.minimum(cid * CH0, TOTC - CHMAX)
        coff = cid * CH0 - start
        nch = jnp.where(cid == 0, CH0, CH1)

        pltpu.sync_copy(edges.at[sid].at[pl.ds(start, CHMAX)], edge_v)
        for t in range(7):
            pltpu.sync_copy(zeros, acc.at[pl.ds(r0 + 80 * t, 80)])
        pltpu.sync_copy(zeros.at[pl.ds(0, 72)], acc.at[pl.ds(r0 + 560, 72)])
        if spmem_table:
            pltpu.sync_copy(table.at[pl.ds(r0, RPT)],
                            maybe_tbl[0].at[pl.ds(r0, RPT)])
            gather_src = maybe_tbl[0]
        else:
            gather_src = table

        def unpack(c, b):
            for j in range(K // 16):
                p = edge_v.at[c + coff][pl.ds(16 * j, 16)]
                sidx.at[b][pl.ds(16 * j, 16)] = lax.bitwise_and(p, 0xFFFF)
                didx.at[b][pl.ds(16 * j, 16)] = lax.shift_right_logical(p, 16)

        plsc.subcore_barrier()

        @pl.when(nch > 0)
        def _():
            unpack(0, 0)
            pltpu.async_copy(gather_src.at[sidx.at[0]], rows0, sem0)

        def body(i, carry):
            a = 2 * i
            unpack(a + 1, 1)
            pltpu.async_copy(gather_src.at[sidx.at[1]], rows1, sem1)
            pltpu.make_async_copy(gather_src.at[sidx.at[0]], rows0, sem0).wait()
            pltpu.sync_copy(rows0, acc.at[didx.at[0]], add=True)

            @pl.when(i < nch // 2 - 1)
            def _():
                unpack(a + 2, 0)
                pltpu.async_copy(gather_src.at[sidx.at[0]], rows0, sem0)

            pltpu.make_async_copy(gather_src.at[sidx.at[1]], rows1, sem1).wait()
            pltpu.sync_copy(rows1, acc.at[didx.at[1]], add=True)
            return carry

        lax.fori_loop(0, nch // 2, body, 0)
        plsc.subcore_barrier()

        pltpu.sync_copy(acc.at[pl.ds(r0, RPT)],
                        out.at[cid].at[pl.ds(r0, RPT)])

    return seg_sum


_seg_sum_128 = _make_seg_sum(F_IN)
_seg_sum_64 = _make_seg_sum(C, spmem_table=True)


def _mlp_body(x_ref, a_ref, w1_ref, b1_ref, w2_ref, o_ref):
    h = x_ref[...] + a_ref[0] + a_ref[1]
    h1 = jnp.dot(h, w1_ref[...], preferred_element_type=jnp.float32)
    h1 = jnp.maximum(h1 + b1_ref[...], 0.0)
    o_ref[...] = jnp.dot(h1, w2_ref[...], preferred_element_type=jnp.float32)


def _log_softmax_body(v_ref, a_ref, b2_ref, o_ref):
    h2 = v_ref[...] + a_ref[0] + a_ref[1] + b2_ref[...]
    m = jnp.max(h2, axis=1, keepdims=True)
    s = jnp.sum(jnp.exp(h2 - m), axis=1, keepdims=True)
    o_ref[...] = h2 - m - jnp.log(s)


_BR = 1000


def _mlp(x, aggr, W1, b1, W2):
    return pl.pallas_call(
        _mlp_body,
        grid=(N // _BR,),
        in_specs=[
            pl.BlockSpec((_BR, F_IN), lambda i: (i, 0)),
            pl.BlockSpec((NC, _BR, F_IN), lambda i: (0, i, 0)),
            pl.BlockSpec((F_IN, H), lambda i: (0, 0)),
            pl.BlockSpec((1, H), lambda i: (0, 0)),
            pl.BlockSpec((H, C), lambda i: (0, 0)),
        ],
        out_specs=pl.BlockSpec((_BR, C), lambda i: (i, 0)),
        out_shape=jax.ShapeDtypeStruct((N, C), jnp.float32),
    )(x, aggr, W1, b1.reshape(1, H), W2)


def _log_softmax(v, aggr, b2):
    return pl.pallas_call(
        _log_softmax_body,
        grid=(N // _BR,),
        in_specs=[
            pl.BlockSpec((_BR, C), lambda i: (i, 0)),
            pl.BlockSpec((NC, _BR, C), lambda i: (0, i, 0)),
            pl.BlockSpec((1, C), lambda i: (0, 0)),
        ],
        out_specs=pl.BlockSpec((_BR, C), lambda i: (i, 0)),
        out_shape=jax.ShapeDtypeStruct((N, C), jnp.float32),
    )(v, aggr, b2.reshape(1, C))


def kernel(x, edge_index, W1, b1, W2, b2):
    src = edge_index[0].astype(jnp.int32)
    dst = edge_index[1].astype(jnp.int32)
    pad = E_PAD - E
    packed = jnp.left_shift(dst, 16) | src
    packed = jnp.concatenate([packed, jnp.full((pad,), DUMMY << 16, jnp.int32)])
    edges = packed.reshape(NS, TOTC, K)

    z128 = jnp.zeros((80, F_IN), jnp.float32)
    z64 = jnp.zeros((80, C), jnp.float32)

    aggr1 = _seg_sum_128(x, edges, z128)
    v = _mlp(x, aggr1, W1, b1, W2)
    v_pad = jnp.concatenate([v, jnp.zeros((NROWS - N, C), jnp.float32)])
    aggr2 = _seg_sum_64(v_pad, edges, z64)
    return _log_softmax(v, aggr2, b2)


# speedup vs baseline: 10.1179x; 1.5574x over previous
import functools

import jax
import jax.numpy as jnp
from jax import lax
from jax.experimental import pallas as pl
from jax.experimental.pallas import tpu as pltpu
from jax.experimental.pallas import tpu_sc as plsc

N = 10000
E = 320000
F_IN = 128
H = 256
C = 64

NC = 2
NS = 16
K = 128
CH0 = 80
CH1 = 80
CHMAX = max(CH0, CH1)
TOTC = CH0 + CH1
E_PAD = NS * (CH0 + CH1) * K
RPT = 632
NROWS = NS * RPT
DUMMY = 10008


def _make_seg_sum(P):
    FH = 64

    @functools.partial(
        pl.kernel,
        out_type=jax.ShapeDtypeStruct((NC, P, NROWS, FH), jnp.float32),
        mesh=plsc.VectorSubcoreMesh(core_axis_name="c", subcore_axis_name="s"),
        scratch_types=[
            pltpu.VMEM((CHMAX, K), jnp.int32),
            pltpu.VMEM((2, K), jnp.int32),
            pltpu.VMEM((2, K), jnp.int32),
            pltpu.VMEM((K, FH), jnp.float32),
            pltpu.VMEM((K, FH), jnp.float32),
            pltpu.VMEM_SHARED((NROWS, FH), jnp.float32),
            pltpu.VMEM_SHARED((NROWS, FH), jnp.float32),
            pltpu.SemaphoreType.DMA,
            pltpu.SemaphoreType.DMA,
        ],
        compiler_params=pltpu.CompilerParams(use_tc_tiling_on_sc=False),
    )
    def seg_sum(table, edges, zeros, out, edge_v, sidx, didx, rows0, rows1,
                acc, tbl, sem0, sem1):
        cid = lax.axis_index("c")
        sid = lax.axis_index("s")
        r0 = sid * RPT
        start = jnp.minimum(cid * CH0, TOTC - CHMAX)
        coff = cid * CH0 - start
        nch = jnp.where(cid == 0, CH0, CH1)

        pltpu.sync_copy(edges.at[sid].at[pl.ds(start, CHMAX)], edge_v)

        def unpack(c, b):
            for j in range(K // 16):
                p = edge_v.at[c + coff][pl.ds(16 * j, 16)]
                sidx.at[b][pl.ds(16 * j, 16)] = lax.bitwise_and(p, 0xFFFF)
                didx.at[b][pl.ds(16 * j, 16)] = lax.shift_right_logical(p, 16)

        for p in range(P):
            pltpu.sync_copy(table.at[p].at[pl.ds(r0, RPT)],
                            tbl.at[pl.ds(r0, RPT)])
            for t in range(7):
                pltpu.sync_copy(zeros, acc.at[pl.ds(r0 + 80 * t, 80)])
            pltpu.sync_copy(zeros.at[pl.ds(0, 72)],
                            acc.at[pl.ds(r0 + 560, 72)])
            plsc.subcore_barrier()

            @pl.when(nch > 0)
            def _():
                unpack(0, 0)
                pltpu.async_copy(tbl.at[sidx.at[0]], rows0, sem0)

            def body(i, carry):
                a = 2 * i
                unpack(a + 1, 1)
                pltpu.async_copy(tbl.at[sidx.at[1]], rows1, sem1)
                pltpu.make_async_copy(tbl.at[sidx.at[0]], rows0, sem0).wait()
                pltpu.sync_copy(rows0, acc.at[didx.at[0]], add=True)

                @pl.when(i < nch // 2 - 1)
                def _():
                    unpack(a + 2, 0)
                    pltpu.async_copy(tbl.at[sidx.at[0]], rows0, sem0)

                pltpu.make_async_copy(tbl.at[sidx.at[1]], rows1, sem1).wait()
                pltpu.sync_copy(rows1, acc.at[didx.at[1]], add=True)
                return carry

            lax.fori_loop(0, nch // 2, body, 0)
            plsc.subcore_barrier()

            pltpu.sync_copy(acc.at[pl.ds(r0, RPT)],
                            out.at[cid].at[p].at[pl.ds(r0, RPT)])

    return seg_sum


_seg_sum_2 = _make_seg_sum(2)
_seg_sum_1 = _make_seg_sum(1)


def _mlp_body(x_ref, a_ref, w1_ref, b1_ref, w2_ref, o_ref):
    h = x_ref[...] + jnp.concatenate(
        [a_ref[0, 0] + a_ref[1, 0], a_ref[0, 1] + a_ref[1, 1]], axis=1)
    h1 = jnp.dot(h, w1_ref[...], preferred_element_type=jnp.float32)
    h1 = jnp.maximum(h1 + b1_ref[...], 0.0)
    o_ref[...] = jnp.dot(h1, w2_ref[...], preferred_element_type=jnp.float32)


def _log_softmax_body(v_ref, a_ref, b2_ref, o_ref):
    h2 = v_ref[...] + a_ref[0, 0] + a_ref[1, 0] + b2_ref[...]
    m = jnp.max(h2, axis=1, keepdims=True)
    s = jnp.sum(jnp.exp(h2 - m), axis=1, keepdims=True)
    o_ref[...] = h2 - m - jnp.log(s)


_BR = 1000


def _mlp(x, aggr, W1, b1, W2):
    return pl.pallas_call(
        _mlp_body,
        grid=(N // _BR,),
        in_specs=[
            pl.BlockSpec((_BR, F_IN), lambda i: (i, 0)),
            pl.BlockSpec((NC, 2, _BR, 64), lambda i: (0, 0, i, 0)),
            pl.BlockSpec((F_IN, H), lambda i: (0, 0)),
            pl.BlockSpec((1, H), lambda i: (0, 0)),
            pl.BlockSpec((H, C), lambda i: (0, 0)),
        ],
        out_specs=pl.BlockSpec((_BR, C), lambda i: (i, 0)),
        out_shape=jax.ShapeDtypeStruct((N, C), jnp.float32),
    )(x, aggr, W1, b1.reshape(1, H), W2)


def _log_softmax(v, aggr, b2):
    return pl.pallas_call(
        _log_softmax_body,
        grid=(N // _BR,),
        in_specs=[
            pl.BlockSpec((_BR, C), lambda i: (i, 0)),
            pl.BlockSpec((NC, 1, _BR, C), lambda i: (0, 0, i, 0)),
            pl.BlockSpec((1, C), lambda i: (0, 0)),
        ],
        out_specs=pl.BlockSpec((_BR, C), lambda i: (i, 0)),
        out_shape=jax.ShapeDtypeStruct((N, C), jnp.float32),
    )(v, aggr, b2.reshape(1, C))


def kernel(x, edge_index, W1, b1, W2, b2):
    src = edge_index[0].astype(jnp.int32)
    dst = edge_index[1].astype(jnp.int32)
    pad = E_PAD - E
    packed = jnp.left_shift(dst, 16) | src
    packed = jnp.concatenate([packed, jnp.full((pad,), DUMMY << 16, jnp.int32)])
    edges = packed.reshape(NS, TOTC, K)

    z64 = jnp.zeros((80, 64), jnp.float32)
    x_pad = jnp.concatenate([x, jnp.zeros((NROWS - N, F_IN), jnp.float32)])
    xh = jnp.stack([x_pad[:, :64], x_pad[:, 64:]])

    aggr1 = _seg_sum_2(xh, edges, z64)
    v = _mlp(x, aggr1, W1, b1, W2)
    v_pad = jnp.concatenate([v, jnp.zeros((NROWS - N, C), jnp.float32)])
    aggr2 = _seg_sum_1(v_pad[None], edges, z64)
    return _log_softmax(v, aggr2, b2)


# speedup vs baseline: 11.4071x; 1.1274x over previous
import functools

import jax
import jax.numpy as jnp
from jax import lax
from jax.experimental import pallas as pl
from jax.experimental.pallas import tpu as pltpu
from jax.experimental.pallas import tpu_sc as plsc

N = 10000
E = 320000
F_IN = 128
H = 256
C = 64

NC = 2
NS = 16
K = 128
CH0 = 80
CH1 = 80
CHMAX = max(CH0, CH1)
TOTC = CH0 + CH1
E_PAD = NS * (CH0 + CH1) * K
RPT = 632
NROWS = NS * RPT
DUMMY = 10008


def _make_seg_sum(P):
    FH = 64

    @functools.partial(
        pl.kernel,
        out_type=jax.ShapeDtypeStruct((NC, P, NROWS, FH), jnp.float32),
        mesh=plsc.VectorSubcoreMesh(core_axis_name="c", subcore_axis_name="s"),
        scratch_types=[
            pltpu.VMEM((CHMAX, K), jnp.int32),
            pltpu.VMEM((2, K), jnp.int32),
            pltpu.VMEM((2, K), jnp.int32),
            pltpu.VMEM((K, FH), jnp.float32),
            pltpu.VMEM((K, FH), jnp.float32),
            pltpu.VMEM_SHARED((NROWS, FH), jnp.float32),
            pltpu.VMEM_SHARED((NROWS, FH), jnp.float32),
            pltpu.SemaphoreType.DMA,
            pltpu.SemaphoreType.DMA,
        ],
        compiler_params=pltpu.CompilerParams(use_tc_tiling_on_sc=False),
    )
    def seg_sum(table, edges, zeros, out, edge_v, sidx, didx, rows0, rows1,
                acc, tbl, sem0, sem1):
        cid = lax.axis_index("c")
        sid = lax.axis_index("s")
        r0 = sid * RPT
        start = jnp.minimum(cid * CH0, TOTC - CHMAX)
        coff = cid * CH0 - start
        nch = jnp.where(cid == 0, CH0, CH1)

        pltpu.sync_copy(edges.at[sid].at[pl.ds(start, CHMAX)], edge_v)

        def unpack(c, b):
            for j in range(K // 16):
                p = edge_v.at[c + coff][pl.ds(16 * j, 16)]
                sidx.at[b][pl.ds(16 * j, 16)] = lax.bitwise_and(p, 0xFFFF)
                didx.at[b][pl.ds(16 * j, 16)] = lax.shift_right_logical(p, 16)

        for p in range(P):
            pltpu.sync_copy(table.at[pl.ds(r0, RPT), pl.ds(FH * p, FH)],
                            tbl.at[pl.ds(r0, RPT)])
            for t in range(7):
                pltpu.sync_copy(zeros, acc.at[pl.ds(r0 + 80 * t, 80)])
            pltpu.sync_copy(zeros.at[pl.ds(0, 72)],
                            acc.at[pl.ds(r0 + 560, 72)])
            plsc.subcore_barrier()

            @pl.when(nch > 0)
            def _():
                unpack(0, 0)
                pltpu.async_copy(tbl.at[sidx.at[0]], rows0, sem0)

            def body(i, carry):
                a = 2 * i
                unpack(a + 1, 1)
                pltpu.async_copy(tbl.at[sidx.at[1]], rows1, sem1)
                pltpu.make_async_copy(tbl.at[sidx.at[0]], rows0, sem0).wait()
                pltpu.sync_copy(rows0, acc.at[didx.at[0]], add=True)

                @pl.when(i < nch // 2 - 1)
                def _():
                    unpack(a + 2, 0)
                    pltpu.async_copy(tbl.at[sidx.at[0]], rows0, sem0)

                pltpu.make_async_copy(tbl.at[sidx.at[1]], rows1, sem1).wait()
                pltpu.sync_copy(rows1, acc.at[didx.at[1]], add=True)
                return carry

            lax.fori_loop(0, nch // 2, body, 0)
            plsc.subcore_barrier()

            pltpu.sync_copy(acc.at[pl.ds(r0, RPT)],
                            out.at[cid].at[p].at[pl.ds(r0, RPT)])

    return seg_sum


_seg_sum_2 = _make_seg_sum(2)
_seg_sum_1 = _make_seg_sum(1)


_EROWS = E // K
_EROWS_PAD = E_PAD // K


def _pack_body(ei_ref, o_ref):
    p = jnp.left_shift(ei_ref[1], 16) | ei_ref[0]
    fill = jnp.full((_EROWS_PAD - _EROWS, K), DUMMY << 16, jnp.int32)
    o_ref[...] = jnp.concatenate([p, fill], axis=0)


def _pack_edges(edge_index):
    out = pl.pallas_call(
        _pack_body,
        out_shape=jax.ShapeDtypeStruct((_EROWS_PAD, K), jnp.int32),
    )(edge_index.astype(jnp.int32).reshape(2, _EROWS, K))
    return out.reshape(NS, TOTC, K)


def _mlp_body(x_ref, a_ref, w1_ref, b1_ref, w2_ref, o_ref):
    h = x_ref[...] + jnp.concatenate(
        [a_ref[0, 0] + a_ref[1, 0], a_ref[0, 1] + a_ref[1, 1]], axis=1)
    h1 = jnp.dot(h, w1_ref[...], preferred_element_type=jnp.float32)
    h1 = jnp.maximum(h1 + b1_ref[...], 0.0)
    o_ref[...] = jnp.dot(h1, w2_ref[...], preferred_element_type=jnp.float32)


def _log_softmax_body(v_ref, a_ref, b2_ref, o_ref):
    h2 = v_ref[...] + a_ref[0, 0] + a_ref[1, 0] + b2_ref[...]
    m = jnp.max(h2, axis=1, keepdims=True)
    s = jnp.sum(jnp.exp(h2 - m), axis=1, keepdims=True)
    o_ref[...] = h2 - m - jnp.log(s)


_BR = 1000


def _mlp(x, aggr, W1, b1, W2):
    return pl.pallas_call(
        _mlp_body,
        grid=(N // _BR,),
        in_specs=[
            pl.BlockSpec((_BR, F_IN), lambda i: (i, 0)),
            pl.BlockSpec((NC, 2, _BR, 64), lambda i: (0, 0, i, 0)),
            pl.BlockSpec((F_IN, H), lambda i: (0, 0)),
            pl.BlockSpec((1, H), lambda i: (0, 0)),
            pl.BlockSpec((H, C), lambda i: (0, 0)),
        ],
        out_specs=pl.BlockSpec((_BR, C), lambda i: (i, 0)),
        out_shape=jax.ShapeDtypeStruct((NROWS, C), jnp.float32),
    )(x, aggr, W1, b1.reshape(1, H), W2)


def _log_softmax(v, aggr, b2):
    return pl.pallas_call(
        _log_softmax_body,
        grid=(N // _BR,),
        in_specs=[
            pl.BlockSpec((_BR, C), lambda i: (i, 0)),
            pl.BlockSpec((NC, 1, _BR, C), lambda i: (0, 0, i, 0)),
            pl.BlockSpec((1, C), lambda i: (0, 0)),
        ],
        out_specs=pl.BlockSpec((_BR, C), lambda i: (i, 0)),
        out_shape=jax.ShapeDtypeStruct((N, C), jnp.float32),
    )(v, aggr, b2.reshape(1, C))


def kernel(x, edge_index, W1, b1, W2, b2):
    edges = _pack_edges(edge_index)
    z64 = jnp.zeros((80, 64), jnp.float32)
    x_pad = jnp.concatenate([x, jnp.zeros((NROWS - N, F_IN), jnp.float32)])

    aggr1 = _seg_sum_2(x_pad, edges, z64)
    v = _mlp(x, aggr1, W1, b1, W2)
    aggr2 = _seg_sum_1(v, edges, z64)
    return _log_softmax(v, aggr2, b2)


# speedup vs baseline: 12.5590x; 1.1010x over previous
import functools

import jax
import jax.numpy as jnp
from jax import lax
from jax.experimental import pallas as pl
from jax.experimental.pallas import tpu as pltpu
from jax.experimental.pallas import tpu_sc as plsc

N = 10000
E = 320000
F_IN = 128
H = 256
C = 64

NC = 2
NS = 16
K = 128
CH0 = 80
CH1 = 80
CHMAX = max(CH0, CH1)
TOTC = CH0 + CH1
E_PAD = NS * (CH0 + CH1) * K
RPT = 632
NROWS = NS * RPT
DUMMY = 10008


def _make_seg_sum(P):
    FH = 64

    @functools.partial(
        pl.kernel,
        out_type=jax.ShapeDtypeStruct((NC, NROWS, 2 * FH), jnp.float32),
        mesh=plsc.VectorSubcoreMesh(core_axis_name="c", subcore_axis_name="s"),
        scratch_types=[
            pltpu.VMEM((CHMAX, K), jnp.int32),
            pltpu.VMEM((2, K), jnp.int32),
            pltpu.VMEM((2, K), jnp.int32),
            pltpu.VMEM((K, FH), jnp.float32),
            pltpu.VMEM((K, FH), jnp.float32),
            pltpu.VMEM_SHARED((NROWS, FH), jnp.float32),
            pltpu.VMEM_SHARED((NROWS, FH), jnp.float32),
            pltpu.SemaphoreType.DMA,
            pltpu.SemaphoreType.DMA,
        ],
        compiler_params=pltpu.CompilerParams(use_tc_tiling_on_sc=False),
    )
    def seg_sum(table, edges, zeros, out, edge_v, sidx, didx, rows0, rows1,
                acc, tbl, sem0, sem1):
        cid = lax.axis_index("c")
        sid = lax.axis_index("s")
        r0 = sid * RPT
        start = jnp.minimum(cid * CH0, TOTC - CHMAX)
        coff = cid * CH0 - start
        nch = jnp.where(cid == 0, CH0, CH1)

        pltpu.sync_copy(edges.at[sid].at[pl.ds(start, CHMAX)], edge_v)

        def unpack(c, b):
            for j in range(K // 16):
                p = edge_v.at[c + coff][pl.ds(16 * j, 16)]
                sidx.at[b][pl.ds(16 * j, 16)] = lax.bitwise_and(p, 0xFFFF)
                didx.at[b][pl.ds(16 * j, 16)] = lax.shift_right_logical(p, 16)

        for p in range(P):
            pltpu.sync_copy(table.at[pl.ds(r0, RPT), pl.ds(FH * p, FH)],
                            tbl.at[pl.ds(r0, RPT)])
            for t in range(7):
                pltpu.sync_copy(zeros, acc.at[pl.ds(r0 + 80 * t, 80)])
            pltpu.sync_copy(zeros.at[pl.ds(0, 72)],
                            acc.at[pl.ds(r0 + 560, 72)])
            plsc.subcore_barrier()

            @pl.when(nch > 0)
            def _():
                unpack(0, 0)
                pltpu.async_copy(tbl.at[sidx.at[0]], rows0, sem0)

            def body(i, carry):
                a = 2 * i
                unpack(a + 1, 1)
                pltpu.async_copy(tbl.at[sidx.at[1]], rows1, sem1)
                pltpu.make_async_copy(tbl.at[sidx.at[0]], rows0, sem0).wait()
                pltpu.sync_copy(rows0, acc.at[didx.at[0]], add=True)

                @pl.when(i < nch // 2 - 1)
                def _():
                    unpack(a + 2, 0)
                    pltpu.async_copy(tbl.at[sidx.at[0]], rows0, sem0)

                pltpu.make_async_copy(tbl.at[sidx.at[1]], rows1, sem1).wait()
                pltpu.sync_copy(rows1, acc.at[didx.at[1]], add=True)
                return carry

            lax.fori_loop(0, nch // 2, body, 0)
            plsc.subcore_barrier()

            pltpu.sync_copy(acc.at[pl.ds(r0, RPT)],
                            out.at[cid].at[pl.ds(r0, RPT),
                                           pl.ds(FH * p, FH)])

    return seg_sum


_seg_sum_2 = _make_seg_sum(2)
_seg_sum_1 = _make_seg_sum(1)


_EROWS = E // K
_EROWS_PAD = E_PAD // K


def _pack_body(ei_ref, o_ref):
    p = jnp.left_shift(ei_ref[1], 16) | ei_ref[0]
    fill = jnp.full((_EROWS_PAD - _EROWS, K), DUMMY << 16, jnp.int32)
    o_ref[...] = jnp.concatenate([p, fill], axis=0)


def _pack_edges(edge_index):
    out = pl.pallas_call(
        _pack_body,
        out_shape=jax.ShapeDtypeStruct((_EROWS_PAD, K), jnp.int32),
    )(edge_index.astype(jnp.int32).reshape(2, _EROWS, K))
    return out.reshape(NS, TOTC, K)


def _mlp_body(x_ref, a_ref, w1_ref, b1_ref, w2_ref, o_ref):
    h = x_ref[...] + a_ref[0] + a_ref[1]
    h1 = jnp.dot(h, w1_ref[...], preferred_element_type=jnp.float32)
    h1 = jnp.maximum(h1 + b1_ref[...], 0.0)
    v = jnp.dot(h1, w2_ref[...], preferred_element_type=jnp.float32)
    o_ref[...] = jnp.concatenate([v, jnp.zeros_like(v)], axis=1)


def _log_softmax_body(v_ref, a_ref, b2_ref, o_ref):
    h2 = (v_ref[...][:, :C] + a_ref[0][:, :C] + a_ref[1][:, :C]
          + b2_ref[...])
    m = jnp.max(h2, axis=1, keepdims=True)
    s = jnp.sum(jnp.exp(h2 - m), axis=1, keepdims=True)
    o_ref[...] = h2 - m - jnp.log(s)


_BR = 1000


def _mlp(x, aggr, W1, b1, W2):
    return pl.pallas_call(
        _mlp_body,
        grid=(N // _BR,),
        in_specs=[
            pl.BlockSpec((_BR, F_IN), lambda i: (i, 0)),
            pl.BlockSpec((NC, _BR, 2 * 64), lambda i: (0, i, 0)),
            pl.BlockSpec((F_IN, H), lambda i: (0, 0)),
            pl.BlockSpec((1, H), lambda i: (0, 0)),
            pl.BlockSpec((H, C), lambda i: (0, 0)),
        ],
        out_specs=pl.BlockSpec((_BR, 2 * C), lambda i: (i, 0)),
        out_shape=jax.ShapeDtypeStruct((NROWS, 2 * C), jnp.float32),
    )(x, aggr, W1, b1.reshape(1, H), W2)


def _log_softmax(v, aggr, b2):
    return pl.pallas_call(
        _log_softmax_body,
        grid=(N // _BR,),
        in_specs=[
            pl.BlockSpec((_BR, 2 * C), lambda i: (i, 0)),
            pl.BlockSpec((NC, _BR, 2 * C), lambda i: (0, i, 0)),
            pl.BlockSpec((1, C), lambda i: (0, 0)),
        ],
        out_specs=pl.BlockSpec((_BR, C), lambda i: (i, 0)),
        out_shape=jax.ShapeDtypeStruct((N, C), jnp.float32),
    )(v, aggr, b2.reshape(1, C))


def kernel(x, edge_index, W1, b1, W2, b2):
    edges = _pack_edges(edge_index)
    z64 = jnp.zeros((80, 64), jnp.float32)
    x_pad = jnp.concatenate([x, jnp.zeros((NROWS - N, F_IN), jnp.float32)])

    aggr1 = _seg_sum_2(x_pad, edges, z64)
    v = _mlp(x, aggr1, W1, b1, W2)
    aggr2 = _seg_sum_1(v, edges, z64)
    return _log_softmax(v, aggr2, b2)
